# baseline (speedup 1.0000x reference)
import functools

import jax
import jax.numpy as jnp
from jax import lax
from jax.experimental import pallas as pl
from jax.experimental.pallas import tpu as pltpu
from jax.experimental.pallas import tpu_sc as plsc

_NUM_CORES = 2
_NUM_SUBCORES = 16
_NW = _NUM_CORES * _NUM_SUBCORES

_SUB = 8
_CHUNK_R = 128
_NSUB = _CHUNK_R // _SUB
_CB = 2048


def _sc_gather(w0_t, wf_t, b0, idx):
  B = idx.shape[0]
  bpw = B // _NW
  mesh = plsc.VectorSubcoreMesh(core_axis_name="c", subcore_axis_name="s")

  @functools.partial(
      pl.kernel,
      mesh=mesh,
      out_type=(
          jax.ShapeDtypeStruct((B, w0_t.shape[1]), jnp.float32),
          jax.ShapeDtypeStruct((B, 128), jnp.float32),
      ),
      scratch_types=[
          pltpu.VMEM((bpw,), jnp.int32),
          pltpu.VMEM((bpw, w0_t.shape[1]), jnp.float32),
          pltpu.VMEM((bpw, wf_t.shape[1]), jnp.float32),
          pltpu.VMEM((bpw, b0.shape[1]), jnp.float32),
          pltpu.VMEM((bpw, 128), jnp.float32),
          pltpu.SemaphoreType.DMA,
          pltpu.SemaphoreType.DMA,
          pltpu.SemaphoreType.DMA,
      ],
      compiler_params=pltpu.CompilerParams(use_tc_tiling_on_sc=False),
  )
  def gather_kernel(w0_hbm, wf_hbm, b0_hbm, idx_hbm,
                    wg_hbm, pk_hbm,
                    idx_v, w_v, wf_v, b_v, pk_v, sem0, sem1, sem2):
    wid = lax.axis_index("s") * _NUM_CORES + lax.axis_index("c")
    base = wid * bpw
    pltpu.sync_copy(idx_hbm.at[pl.ds(base, bpw)], idx_v)
    cp0 = pltpu.async_copy(w0_hbm.at[idx_v], w_v, sem0)
    cp1 = pltpu.async_copy(wf_hbm.at[idx_v], wf_v, sem1)
    cp2 = pltpu.async_copy(b0_hbm.at[idx_v], b_v, sem2)
    cp0.wait()
    cp1.wait()
    cp2.wait()
    zeros16 = jnp.zeros((16,), jnp.float32)
    for i in range(bpw):
      pk_v[i, 0:16] = wf_v[i, :]
      pk_v[i, 16:32] = b_v[i, :]
      for t in range(2, 8):
        pk_v[i, t * 16:(t + 1) * 16] = zeros16
    pltpu.sync_copy(w_v, wg_hbm.at[pl.ds(base, bpw)])
    pltpu.sync_copy(pk_v, pk_hbm.at[pl.ds(base, bpw)])

  return gather_kernel(w0_t, wf_t, b0, idx)


def _dense_body(x_ref, wg_ref, pk_ref, out_ref, wbd_ref, s_ref, bc_ref):
  cb = pl.program_id(1)
  W = _SUB * 16

  @pl.when(cb == 0)
  def _build():
    rr = lax.broadcasted_iota(jnp.int32, (W, W), 0)
    cc = lax.broadcasted_iota(jnp.int32, (W, W), 1)
    msk = jnp.where(rr // 16 == cc // 16, 1.0, 0.0).astype(jnp.float32)
    pr = lax.broadcasted_iota(jnp.int32, (W, _SUB), 0)
    pc = lax.broadcasted_iota(jnp.int32, (W, _SUB), 1)
    pmat = jnp.where(pr // 16 == pc, 1.0, 0.0).astype(jnp.float32)
    mr = lax.broadcasted_iota(jnp.int32, (W, 256), 0)
    mc = lax.broadcasted_iota(jnp.int32, (W, 256), 1)
    mask_e = jnp.where(mc % 16 == mr % 16, 1.0, 0.0).astype(jnp.float32)
    gr = lax.broadcasted_iota(jnp.int32, (256, W), 0)
    gc = lax.broadcasted_iota(jnp.int32, (256, W), 1)
    g2 = jnp.where(gr // 16 == gc % 16, 1.0, 0.0).astype(jnp.float32)
    hr = lax.broadcasted_iota(jnp.int32, (16, W), 0)
    hc = lax.broadcasted_iota(jnp.int32, (16, W), 1)
    g3 = jnp.where(hc % 16 == hr, 1.0, 0.0).astype(jnp.float32)

    wfblk = pk_ref[:, 0:16]
    bblk = pk_ref[:, 16:32]
    e_wf = lax.dot(wfblk, g3, preferred_element_type=jnp.float32)
    e_b = lax.dot(bblk, g3, preferred_element_type=jnp.float32)
    et_b = lax.transpose(e_b, (1, 0))
    for j in range(_NSUB):
      w8 = wg_ref[j * _SUB:(j + 1) * _SUB, :]
      t1 = lax.dot(pmat, w8, preferred_element_type=jnp.float32) * mask_e
      wbd_ref[j] = (lax.dot(t1, g2, preferred_element_type=jnp.float32)
                    * msk).astype(jnp.bfloat16)
      s_ref[j] = jnp.where(rr - j * _SUB == cc // 16, e_wf,
                           0.0).astype(jnp.bfloat16)
      bm = jnp.where(cc - j * _SUB == rr // 16, et_b, 0.0)
      bc_ref[j * W:(j + 1) * W, :] = jnp.sum(bm, axis=1, keepdims=True)

  inv_sqrt2 = 0.7071067811865476
  x2 = x_ref[...].reshape(_CHUNK_R * 16, _CB)
  acc = jnp.zeros((_CHUNK_R, _CB), jnp.float32)
  for j in range(_NSUB):
    xg = x2[j * W:(j + 1) * W, :]
    h = lax.dot(wbd_ref[j], xg, preferred_element_type=jnp.float32)
    h = h + bc_ref[j * W:(j + 1) * W, :]
    h = 0.5 * h * (1.0 + lax.erf(h * inv_sqrt2))
    acc = acc + lax.dot(s_ref[j], h.astype(jnp.bfloat16),
                        preferred_element_type=jnp.float32)
  out_ref[...] = lax.transpose(acc, (1, 0))


def _dense(xq, wg, pk, C, R):
  n_chunks = R // _CHUNK_R
  n_cb = C // _CB
  grid = (n_chunks, n_cb)
  return pl.pallas_call(
      _dense_body,
      grid=grid,
      in_specs=[
          pl.BlockSpec((_CHUNK_R, 16, _CB), lambda k, cb: (k, 0, cb)),
          pl.BlockSpec((_CHUNK_R, 256), lambda k, cb: (k, 0)),
          pl.BlockSpec((_CHUNK_R, 128), lambda k, cb: (k, 0)),
      ],
      out_specs=pl.BlockSpec((_CB, _CHUNK_R), lambda k, cb: (cb, k)),
      out_shape=jax.ShapeDtypeStruct((C, R), jnp.float32),
      scratch_shapes=[
          pltpu.VMEM((_NSUB, _SUB * 16, _SUB * 16), jnp.bfloat16),
          pltpu.VMEM((_NSUB, _SUB * 16, _CHUNK_R), jnp.bfloat16),
          pltpu.VMEM((_CHUNK_R * 16, 1), jnp.float32),
      ],
      compiler_params=pltpu.CompilerParams(
          dimension_semantics=("arbitrary", "arbitrary"),
      ),
  )(xq, wg, pk)


def kernel(cell_region_embedding, regions_oi, W0, b0, Wf):
  C, R, D = cell_region_embedding.shape
  N = W0.shape[0]
  idx = regions_oi.astype(jnp.int32)

  w0_t = W0.reshape(N, D * D)
  wf_t = Wf[:, :, 0]
  wg, pk = _sc_gather(w0_t, wf_t, b0, idx)

  xq = jnp.transpose(cell_region_embedding, (1, 2, 0)).astype(jnp.bfloat16)
  return _dense(xq, wg, pk, C, R)


# speedup vs baseline: 1.1020x; 1.1020x over previous
import functools

import jax
import jax.numpy as jnp
from jax import lax
from jax.experimental import pallas as pl
from jax.experimental.pallas import tpu as pltpu
from jax.experimental.pallas import tpu_sc as plsc

_NUM_CORES = 2
_NUM_SUBCORES = 16
_NW = _NUM_CORES * _NUM_SUBCORES

_SUB = 8
_CHUNK_R = 128
_NSUB = _CHUNK_R // _SUB
_CB = 2048


def _sc_gather(w0_t, wf_t, b0, idx):
  B = idx.shape[0]
  bpw = B // _NW
  mesh = plsc.VectorSubcoreMesh(core_axis_name="c", subcore_axis_name="s")

  @functools.partial(
      pl.kernel,
      mesh=mesh,
      out_type=(
          jax.ShapeDtypeStruct((B, w0_t.shape[1]), jnp.float32),
          jax.ShapeDtypeStruct((B, 128), jnp.float32),
      ),
      scratch_types=[
          pltpu.VMEM((bpw,), jnp.int32),
          pltpu.VMEM((bpw, w0_t.shape[1]), jnp.float32),
          pltpu.VMEM((bpw, wf_t.shape[1]), jnp.float32),
          pltpu.VMEM((bpw, b0.shape[1]), jnp.float32),
          pltpu.VMEM((bpw, 128), jnp.float32),
          pltpu.SemaphoreType.DMA,
          pltpu.SemaphoreType.DMA,
          pltpu.SemaphoreType.DMA,
      ],
      compiler_params=pltpu.CompilerParams(use_tc_tiling_on_sc=False),
  )
  def gather_kernel(w0_hbm, wf_hbm, b0_hbm, idx_hbm,
                    wg_hbm, pk_hbm,
                    idx_v, w_v, wf_v, b_v, pk_v, sem0, sem1, sem2):
    wid = lax.axis_index("s") * _NUM_CORES + lax.axis_index("c")
    base = wid * bpw
    pltpu.sync_copy(idx_hbm.at[pl.ds(base, bpw)], idx_v)
    cp0 = pltpu.async_copy(w0_hbm.at[idx_v], w_v, sem0)
    cp1 = pltpu.async_copy(wf_hbm.at[idx_v], wf_v, sem1)
    cp2 = pltpu.async_copy(b0_hbm.at[idx_v], b_v, sem2)
    cp0.wait()
    cp1.wait()
    cp2.wait()
    zeros16 = jnp.zeros((16,), jnp.float32)
    for i in range(bpw):
      pk_v[i, 0:16] = wf_v[i, :]
      pk_v[i, 16:32] = b_v[i, :]
      for t in range(2, 8):
        pk_v[i, t * 16:(t + 1) * 16] = zeros16
    pltpu.sync_copy(w_v, wg_hbm.at[pl.ds(base, bpw)])
    pltpu.sync_copy(pk_v, pk_hbm.at[pl.ds(base, bpw)])

  return gather_kernel(w0_t, wf_t, b0, idx)


def _dense_body(x_ref, wg_ref, pk_ref, out_ref, wbd_ref, s_ref, bc_ref):
  cb = pl.program_id(1)
  W = _SUB * 16

  @pl.when(cb == 0)
  def _build():
    rr = lax.broadcasted_iota(jnp.int32, (W, W), 0)
    cc = lax.broadcasted_iota(jnp.int32, (W, W), 1)
    msk = jnp.where(rr // 16 == cc // 16, 1.0, 0.0).astype(jnp.float32)
    pr = lax.broadcasted_iota(jnp.int32, (W, _SUB), 0)
    pc = lax.broadcasted_iota(jnp.int32, (W, _SUB), 1)
    pmat = jnp.where(pr // 16 == pc, 1.0, 0.0).astype(jnp.float32)
    mr = lax.broadcasted_iota(jnp.int32, (W, 256), 0)
    mc = lax.broadcasted_iota(jnp.int32, (W, 256), 1)
    mask_e = jnp.where(mc % 16 == mr % 16, 1.0, 0.0).astype(jnp.float32)
    gr = lax.broadcasted_iota(jnp.int32, (256, W), 0)
    gc = lax.broadcasted_iota(jnp.int32, (256, W), 1)
    g2 = jnp.where(gr // 16 == gc % 16, 1.0, 0.0).astype(jnp.float32)
    hr = lax.broadcasted_iota(jnp.int32, (16, W), 0)
    hc = lax.broadcasted_iota(jnp.int32, (16, W), 1)
    g3 = jnp.where(hc % 16 == hr, 1.0, 0.0).astype(jnp.float32)

    wfblk = pk_ref[:, 0:16]
    bblk = pk_ref[:, 16:32]
    e_wf = lax.dot(wfblk, g3, preferred_element_type=jnp.float32)
    e_b = lax.dot(bblk, g3, preferred_element_type=jnp.float32)
    et_b = lax.transpose(e_b, (1, 0))
    for j in range(_NSUB):
      w8 = wg_ref[j * _SUB:(j + 1) * _SUB, :]
      t1 = lax.dot(pmat, w8, preferred_element_type=jnp.float32) * mask_e
      wbd_ref[j] = lax.dot(t1, g2, preferred_element_type=jnp.float32) * msk
      s_ref[j] = jnp.where(rr - j * _SUB == cc // 16, e_wf, 0.0)
      bm = jnp.where(cc - j * _SUB == rr // 16, et_b, 0.0)
      bc_ref[j * W:(j + 1) * W, :] = jnp.sum(bm, axis=1, keepdims=True)

  inv_sqrt2 = 0.7071067811865476
  x2 = x_ref[...].reshape(_CHUNK_R * 16, _CB)
  acc = jnp.zeros((_CHUNK_R, _CB), jnp.float32)
  for j in range(_NSUB):
    xg = x2[j * W:(j + 1) * W, :]
    h = lax.dot(wbd_ref[j], xg, preferred_element_type=jnp.float32)
    h = h + bc_ref[j * W:(j + 1) * W, :]
    h = 0.5 * h * (1.0 + lax.erf(h * inv_sqrt2))
    acc = acc + lax.dot(s_ref[j], h, preferred_element_type=jnp.float32)
  out_ref[...] = lax.transpose(acc, (1, 0))


def _dense(xq, wg, pk, C, R):
  n_chunks = R // _CHUNK_R
  n_cb = C // _CB
  grid = (n_chunks, n_cb)
  return pl.pallas_call(
      _dense_body,
      grid=grid,
      in_specs=[
          pl.BlockSpec((_CHUNK_R, 16, _CB), lambda k, cb: (k, 0, cb)),
          pl.BlockSpec((_CHUNK_R, 256), lambda k, cb: (k, 0)),
          pl.BlockSpec((_CHUNK_R, 128), lambda k, cb: (k, 0)),
      ],
      out_specs=pl.BlockSpec((_CB, _CHUNK_R), lambda k, cb: (cb, k)),
      out_shape=jax.ShapeDtypeStruct((C, R), jnp.float32),
      scratch_shapes=[
          pltpu.VMEM((_NSUB, _SUB * 16, _SUB * 16), jnp.float32),
          pltpu.VMEM((_NSUB, _SUB * 16, _CHUNK_R), jnp.float32),
          pltpu.VMEM((_CHUNK_R * 16, 1), jnp.float32),
      ],
      compiler_params=pltpu.CompilerParams(
          dimension_semantics=("arbitrary", "arbitrary"),
      ),
  )(xq, wg, pk)


def kernel(cell_region_embedding, regions_oi, W0, b0, Wf):
  C, R, D = cell_region_embedding.shape
  N = W0.shape[0]
  idx = regions_oi.astype(jnp.int32)

  w0_t = W0.reshape(N, D * D)
  wf_t = Wf[:, :, 0]
  wg, pk = _sc_gather(w0_t, wf_t, b0, idx)

  xq = jnp.transpose(cell_region_embedding, (1, 2, 0))
  return _dense(xq, wg, pk, C, R)
